# baseline (device time: 345288 ns/iter reference)
import jax
import jax.numpy as jnp
from jax import lax
from jax.experimental import pallas as pl
from jax.experimental.pallas import tpu as pltpu

N_DEV = 4
SCALE = 0.08838834764831843
NEG_BIG = -1e30


def kernel(x, Wq, Wo, K_ext, V_ext):
    _, sq, d = x.shape
    _, skv, hq, dh = K_ext.shape
    assert hq * dh == d and skv % 4 == 0
    half = skv // 2
    sub = half // 2

    x2 = x.reshape(sq, d)
    k3 = jnp.transpose(K_ext[0], (1, 0, 2))
    v3 = jnp.transpose(V_ext[0], (1, 0, 2))

    def body(x_ref, wq_ref, wo_hbm, k_hbm, v_hbm, out_ref,
             k_buf, v_buf, q3_ref, acc_ref, m_ref, l_ref, wo_ref,
             local_sems, send_sems, recv_sems, credit_a, credit_b):
        my = lax.axis_index("i")
        left = lax.rem(my + N_DEV - 1, N_DEV)
        right = lax.rem(my + 1, N_DEV)

        def compute_slot(s, w_off, rows):
            def chunk_body(i, _):
                head = i // 2
                r0 = (i % 2) * half + w_off
                qh = q3_ref[head]
                kh = k_buf[s, head, pl.ds(r0, rows), :]
                vh = v_buf[s, head, pl.ds(r0, rows), :]
                scores = lax.dot_general(
                    qh, kh, (((1,), (1,)), ((), ())),
                    preferred_element_type=jnp.float32) * SCALE
                mj = jnp.max(scores, axis=1, keepdims=True)
                m_prev = m_ref[head]
                m_new = jnp.maximum(m_prev, mj)
                alpha = jnp.exp(m_prev - m_new)
                p = jnp.exp(scores - m_new)
                l_new = (l_ref[head] * alpha
                         + jnp.sum(p, axis=1, keepdims=True))
                acc_new = (acc_ref[head] * alpha
                           + jnp.dot(p, vh,
                                     preferred_element_type=jnp.float32))
                m_ref[head] = m_new
                l_ref[head] = l_new
                acc_ref[head] = acc_new
                return 0

            lax.fori_loop(0, hq * 2, chunk_body, 0)

        def stream_rdma(src, dst, r0, rows, sem_idx, target):
            return pltpu.make_async_remote_copy(
                src_ref=src.at[:, pl.ds(r0, rows), :],
                dst_ref=dst.at[:, pl.ds(r0, rows), :],
                send_sem=send_sems.at[sem_idx],
                recv_sem=recv_sems.at[sem_idx],
                device_id=(target,),
                device_id_type=pl.DeviceIdType.MESH)

        cpk = pltpu.make_async_copy(k_hbm, k_buf.at[0], local_sems.at[0])
        cpv = pltpu.make_async_copy(v_hbm, v_buf.at[0], local_sems.at[1])
        cpk.start()
        cpv.start()

        barrier = pltpu.get_barrier_semaphore()
        for nbr in (left, right):
            pl.semaphore_signal(barrier, inc=1, device_id=(nbr,),
                                device_id_type=pl.DeviceIdType.MESH)
        pl.semaphore_wait(barrier, 2)

        h0 = []
        for hbm, buf, sem0 in ((k_hbm, k_buf, 0), (v_hbm, v_buf, 2)):
            h0.append(stream_rdma(hbm, buf.at[1], 0, half, sem0, right))
            h0.append(stream_rdma(hbm, buf.at[1], half, half, sem0 + 1,
                                  left))
        for r in h0:
            r.start()

        q = jnp.dot(x_ref[...], wq_ref[...],
                    preferred_element_type=jnp.float32)
        for hd in range(hq):
            q3_ref[hd] = q[:, hd * dh:(hd + 1) * dh]
        acc_ref[...] = jnp.zeros((hq, sq, dh), jnp.float32)
        m_ref[...] = jnp.full((hq, sq, 1), NEG_BIG, jnp.float32)
        l_ref[...] = jnp.zeros((hq, sq, 1), jnp.float32)

        cpk.wait()
        cpv.wait()

        compute_slot(0, 0, half)
        for r in h0:
            r.wait()
        pl.semaphore_signal(credit_a, inc=1, device_id=(left,),
                            device_id_type=pl.DeviceIdType.MESH)
        pl.semaphore_signal(credit_b, inc=1, device_id=(right,),
                            device_id_type=pl.DeviceIdType.MESH)

        pl.semaphore_wait(credit_a, 1)
        pl.semaphore_wait(credit_b, 1)
        h1 = []
        for buf, sem0 in ((k_buf, 0), (v_buf, 2)):
            h1.append(stream_rdma(buf.at[1], buf.at[0], 0, half, sem0,
                                  right))
            h1.append(stream_rdma(buf.at[1], buf.at[0], half, half,
                                  sem0 + 1, left))
        for r in h1:
            r.start()
        compute_slot(1, 0, half)
        for r in h1:
            r.wait()
        pl.semaphore_signal(credit_a, inc=1, device_id=(left,),
                            device_id_type=pl.DeviceIdType.MESH)
        pl.semaphore_signal(credit_b, inc=1, device_id=(right,),
                            device_id_type=pl.DeviceIdType.MESH)

        pl.semaphore_wait(credit_a, 1)
        pl.semaphore_wait(credit_b, 1)
        waves = [[], []]
        for buf, sem0 in ((k_buf, 0), (v_buf, 2)):
            for w in range(2):
                waves[w].append(stream_rdma(
                    buf.at[0], buf.at[1], w * sub, sub,
                    sem0 + 4 * w, right))
                waves[w].append(stream_rdma(
                    buf.at[0], buf.at[1], half + w * sub, sub,
                    sem0 + 1 + 4 * w, left))
        for w in range(2):
            for r in waves[w]:
                r.start()
        compute_slot(0, 0, half)
        for w in range(2):
            for r in waves[w]:
                r.wait_send()

        cpw = pltpu.make_async_copy(wo_hbm, wo_ref, local_sems.at[0])
        cpw.start()

        for w in range(2):
            for r in waves[w]:
                r.wait_recv()
            compute_slot(1, w * sub, sub)

        attn = jnp.concatenate(
            [acc_ref[hd] / l_ref[hd] for hd in range(hq)], axis=1)
        cpw.wait()
        out_ref[...] = jnp.dot(attn, wo_ref[...],
                               preferred_element_type=jnp.float32)

    out = pl.pallas_call(
        body,
        out_shape=jax.ShapeDtypeStruct((sq, d), jnp.float32),
        in_specs=[
            pl.BlockSpec(memory_space=pltpu.VMEM),
            pl.BlockSpec(memory_space=pltpu.VMEM),
            pl.BlockSpec(memory_space=pl.ANY),
            pl.BlockSpec(memory_space=pl.ANY),
            pl.BlockSpec(memory_space=pl.ANY),
        ],
        out_specs=pl.BlockSpec(memory_space=pltpu.VMEM),
        scratch_shapes=[
            pltpu.VMEM((2, hq, skv, dh), jnp.float32),
            pltpu.VMEM((2, hq, skv, dh), jnp.float32),
            pltpu.VMEM((hq, sq, dh), jnp.float32),
            pltpu.VMEM((hq, sq, dh), jnp.float32),
            pltpu.VMEM((hq, sq, 1), jnp.float32),
            pltpu.VMEM((hq, sq, 1), jnp.float32),
            pltpu.VMEM((d, d), jnp.float32),
            pltpu.SemaphoreType.DMA((2,)),
            pltpu.SemaphoreType.DMA((8,)),
            pltpu.SemaphoreType.DMA((8,)),
            pltpu.SemaphoreType.REGULAR,
            pltpu.SemaphoreType.REGULAR,
        ],
        compiler_params=pltpu.CompilerParams(
            collective_id=0, vmem_limit_bytes=67043328),
    )(x2, Wq, Wo, k3, v3)
    return out.reshape(1, sq, d)


# device time: 198365 ns/iter; 1.7407x vs baseline; 1.7407x over previous
import jax
import jax.numpy as jnp
from jax import lax
from jax.experimental import pallas as pl
from jax.experimental.pallas import tpu as pltpu

N_DEV = 4
SCALE = 0.08838834764831843
NEG_BIG = -1e30
KV_CHUNK = 1024


def kernel(x, Wq, Wo, K_ext, V_ext):
    _, sq, d = x.shape
    _, skv, hq, dh = K_ext.shape
    assert hq * dh == d and skv % KV_CHUNK == 0
    n_chunks = skv // KV_CHUNK
    half = skv // 2

    x2 = x.reshape(sq, d)
    k3 = jnp.transpose(K_ext[0], (1, 0, 2)).astype(jnp.bfloat16)
    v3 = jnp.transpose(V_ext[0], (1, 0, 2)).astype(jnp.bfloat16)

    def body(x_ref, wq_ref, wo_hbm, k_hbm, v_hbm, out_ref,
             k_buf, v_buf, q3_ref, acc_ref, m_ref, l_ref, wo_ref,
             local_sems, send_sems, recv_sems, credit_a, credit_b):
        my = lax.axis_index("i")
        left = lax.rem(my + N_DEV - 1, N_DEV)
        right = lax.rem(my + 1, N_DEV)

        cpk = pltpu.make_async_copy(k_hbm, k_buf.at[0], local_sems.at[0])
        cpv = pltpu.make_async_copy(v_hbm, v_buf.at[0], local_sems.at[1])
        cpk.start()
        cpv.start()

        barrier = pltpu.get_barrier_semaphore()
        for nbr in (left, right):
            pl.semaphore_signal(barrier, inc=1, device_id=(nbr,),
                                device_id_type=pl.DeviceIdType.MESH)
        pl.semaphore_wait(barrier, 2)

        q = jnp.dot(x_ref[...], wq_ref[...],
                    preferred_element_type=jnp.float32)
        for hd in range(hq):
            q3_ref[hd] = q[:, hd * dh:(hd + 1) * dh].astype(jnp.bfloat16)
        acc_ref[...] = jnp.zeros((hq, sq, dh), jnp.float32)
        m_ref[...] = jnp.full((hq, sq, 1), NEG_BIG, jnp.float32)
        l_ref[...] = jnp.zeros((hq, sq, 1), jnp.float32)

        cpk.wait()
        cpv.wait()

        for h in range(N_DEV):
            s = h % 2
            if h < N_DEV - 1:
                if h >= 1:
                    pl.semaphore_wait(credit_a, 1)
                    pl.semaphore_wait(credit_b, 1)
                rdmas = []
                for buf, sem0 in ((k_buf, 0), (v_buf, 2)):
                    rdmas.append(pltpu.make_async_remote_copy(
                        src_ref=buf.at[s, :, pl.ds(0, half), :],
                        dst_ref=buf.at[1 - s, :, pl.ds(0, half), :],
                        send_sem=send_sems.at[sem0],
                        recv_sem=recv_sems.at[sem0],
                        device_id=(right,),
                        device_id_type=pl.DeviceIdType.MESH))
                    rdmas.append(pltpu.make_async_remote_copy(
                        src_ref=buf.at[s, :, pl.ds(half, half), :],
                        dst_ref=buf.at[1 - s, :, pl.ds(half, half), :],
                        send_sem=send_sems.at[sem0 + 1],
                        recv_sem=recv_sems.at[sem0 + 1],
                        device_id=(left,),
                        device_id_type=pl.DeviceIdType.MESH))
                for r in rdmas:
                    r.start()

            def chunk_body(i, _, s=s):
                head = i // n_chunks
                r0 = (i % n_chunks) * KV_CHUNK
                qh = q3_ref[head]
                kh = k_buf[s, head, pl.ds(r0, KV_CHUNK), :]
                vh = v_buf[s, head, pl.ds(r0, KV_CHUNK), :]
                scores = lax.dot_general(
                    qh, kh, (((1,), (1,)), ((), ())),
                    preferred_element_type=jnp.float32) * SCALE
                mj = jnp.max(scores, axis=1, keepdims=True)
                m_prev = m_ref[head]
                m_new = jnp.maximum(m_prev, mj)
                alpha = jnp.exp(m_prev - m_new)
                p = jnp.exp(scores - m_new).astype(jnp.bfloat16)
                l_new = (l_ref[head] * alpha
                         + jnp.sum(p.astype(jnp.float32), axis=1,
                                   keepdims=True))
                acc_new = (acc_ref[head] * alpha
                           + jnp.dot(p, vh,
                                     preferred_element_type=jnp.float32))
                m_ref[head] = m_new
                l_ref[head] = l_new
                acc_ref[head] = acc_new
                return 0

            lax.fori_loop(0, hq * n_chunks, chunk_body, 0)

            if h < N_DEV - 1:
                for r in rdmas:
                    r.wait()
                if h < N_DEV - 2:
                    pl.semaphore_signal(credit_a, inc=1, device_id=(left,),
                                        device_id_type=pl.DeviceIdType.MESH)
                    pl.semaphore_signal(credit_b, inc=1, device_id=(right,),
                                        device_id_type=pl.DeviceIdType.MESH)

            if h == N_DEV - 2:
                cpw = pltpu.make_async_copy(wo_hbm, wo_ref,
                                            local_sems.at[0])
                cpw.start()

        attn = jnp.concatenate(
            [acc_ref[hd] / l_ref[hd] for hd in range(hq)], axis=1)
        cpw.wait()
        out_ref[...] = jnp.dot(attn, wo_ref[...],
                               preferred_element_type=jnp.float32)

    out = pl.pallas_call(
        body,
        out_shape=jax.ShapeDtypeStruct((sq, d), jnp.float32),
        in_specs=[
            pl.BlockSpec(memory_space=pltpu.VMEM),
            pl.BlockSpec(memory_space=pltpu.VMEM),
            pl.BlockSpec(memory_space=pl.ANY),
            pl.BlockSpec(memory_space=pl.ANY),
            pl.BlockSpec(memory_space=pl.ANY),
        ],
        out_specs=pl.BlockSpec(memory_space=pltpu.VMEM),
        scratch_shapes=[
            pltpu.VMEM((2, hq, skv, dh), jnp.bfloat16),
            pltpu.VMEM((2, hq, skv, dh), jnp.bfloat16),
            pltpu.VMEM((hq, sq, dh), jnp.bfloat16),
            pltpu.VMEM((hq, sq, dh), jnp.float32),
            pltpu.VMEM((hq, sq, 1), jnp.float32),
            pltpu.VMEM((hq, sq, 1), jnp.float32),
            pltpu.VMEM((d, d), jnp.float32),
            pltpu.SemaphoreType.DMA((2,)),
            pltpu.SemaphoreType.DMA((4,)),
            pltpu.SemaphoreType.DMA((4,)),
            pltpu.SemaphoreType.REGULAR,
            pltpu.SemaphoreType.REGULAR,
        ],
        compiler_params=pltpu.CompilerParams(
            collective_id=0, vmem_limit_bytes=67043328),
    )(x2, Wq, Wo, k3, v3)
    return out.reshape(1, sq, d)


# device time: 186961 ns/iter; 1.8468x vs baseline; 1.0610x over previous
import jax
import jax.numpy as jnp
from jax import lax
from jax.experimental import pallas as pl
from jax.experimental.pallas import tpu as pltpu

N_DEV = 4
SCALE = 0.08838834764831843
NEG_BIG = -1e30
KV_CHUNK = 2048


def kernel(x, Wq, Wo, K_ext, V_ext):
    _, sq, d = x.shape
    _, skv, hq, dh = K_ext.shape
    assert hq * dh == d and skv % KV_CHUNK == 0
    n_chunks = skv // KV_CHUNK
    half = skv // 2

    x2 = x.reshape(sq, d)
    k3 = jnp.transpose(K_ext[0], (1, 0, 2)).astype(jnp.bfloat16)
    v3 = jnp.transpose(V_ext[0], (1, 0, 2)).astype(jnp.bfloat16)

    def body(x_ref, wq_ref, wo_hbm, k_hbm, v_hbm, out_ref,
             k_buf, v_buf, q3_ref, acc_ref, m_ref, l_ref, wo_ref,
             local_sems, send_sems, recv_sems, credit_a, credit_b):
        my = lax.axis_index("i")
        left = lax.rem(my + N_DEV - 1, N_DEV)
        right = lax.rem(my + 1, N_DEV)

        cpk = pltpu.make_async_copy(k_hbm, k_buf.at[0], local_sems.at[0])
        cpv = pltpu.make_async_copy(v_hbm, v_buf.at[0], local_sems.at[1])
        cpk.start()
        cpv.start()

        barrier = pltpu.get_barrier_semaphore()
        for nbr in (left, right):
            pl.semaphore_signal(barrier, inc=1, device_id=(nbr,),
                                device_id_type=pl.DeviceIdType.MESH)
        pl.semaphore_wait(barrier, 2)

        q = jnp.dot(x_ref[...], wq_ref[...],
                    preferred_element_type=jnp.float32)
        for hd in range(hq):
            q3_ref[hd] = q[:, hd * dh:(hd + 1) * dh].astype(jnp.bfloat16)
        acc_ref[...] = jnp.zeros((hq, sq, dh), jnp.float32)
        m_ref[...] = jnp.full((hq, sq, 1), NEG_BIG, jnp.float32)
        l_ref[...] = jnp.zeros((hq, sq, 1), jnp.float32)

        cpk.wait()
        cpv.wait()

        for h in range(N_DEV):
            s = h % 2
            if h < N_DEV - 1:
                if h >= 1:
                    pl.semaphore_wait(credit_a, 1)
                    pl.semaphore_wait(credit_b, 1)
                rdmas = []
                for buf, sem0 in ((k_buf, 0), (v_buf, 2)):
                    rdmas.append(pltpu.make_async_remote_copy(
                        src_ref=buf.at[s, :, pl.ds(0, half), :],
                        dst_ref=buf.at[1 - s, :, pl.ds(0, half), :],
                        send_sem=send_sems.at[sem0],
                        recv_sem=recv_sems.at[sem0],
                        device_id=(right,),
                        device_id_type=pl.DeviceIdType.MESH))
                    rdmas.append(pltpu.make_async_remote_copy(
                        src_ref=buf.at[s, :, pl.ds(half, half), :],
                        dst_ref=buf.at[1 - s, :, pl.ds(half, half), :],
                        send_sem=send_sems.at[sem0 + 1],
                        recv_sem=recv_sems.at[sem0 + 1],
                        device_id=(left,),
                        device_id_type=pl.DeviceIdType.MESH))
                for r in rdmas:
                    r.start()

            def chunk_body(i, _, s=s):
                head = i // n_chunks
                r0 = (i % n_chunks) * KV_CHUNK
                qh = q3_ref[head]
                kh = k_buf[s, head, pl.ds(r0, KV_CHUNK), :]
                vh = v_buf[s, head, pl.ds(r0, KV_CHUNK), :]
                scores = lax.dot_general(
                    qh, kh, (((1,), (1,)), ((), ())),
                    preferred_element_type=jnp.float32) * SCALE
                mj = jnp.max(scores, axis=1, keepdims=True)
                m_prev = m_ref[head]
                m_new = jnp.maximum(m_prev, mj)
                alpha = jnp.exp(m_prev - m_new)
                p = jnp.exp(scores - m_new).astype(jnp.bfloat16)
                l_new = (l_ref[head] * alpha
                         + jnp.sum(p.astype(jnp.float32), axis=1,
                                   keepdims=True))
                acc_new = (acc_ref[head] * alpha
                           + jnp.dot(p, vh,
                                     preferred_element_type=jnp.float32))
                m_ref[head] = m_new
                l_ref[head] = l_new
                acc_ref[head] = acc_new
                return 0

            lax.fori_loop(0, hq * n_chunks, chunk_body, 0)

            if h < N_DEV - 1:
                for r in rdmas:
                    r.wait()
                if h < N_DEV - 2:
                    pl.semaphore_signal(credit_a, inc=1, device_id=(left,),
                                        device_id_type=pl.DeviceIdType.MESH)
                    pl.semaphore_signal(credit_b, inc=1, device_id=(right,),
                                        device_id_type=pl.DeviceIdType.MESH)

            if h == N_DEV - 2:
                cpw = pltpu.make_async_copy(wo_hbm, wo_ref,
                                            local_sems.at[0])
                cpw.start()

        attn = jnp.concatenate(
            [acc_ref[hd] / l_ref[hd] for hd in range(hq)], axis=1)
        cpw.wait()
        out_ref[...] = jnp.dot(attn, wo_ref[...],
                               preferred_element_type=jnp.float32)

    out = pl.pallas_call(
        body,
        out_shape=jax.ShapeDtypeStruct((sq, d), jnp.float32),
        in_specs=[
            pl.BlockSpec(memory_space=pltpu.VMEM),
            pl.BlockSpec(memory_space=pltpu.VMEM),
            pl.BlockSpec(memory_space=pl.ANY),
            pl.BlockSpec(memory_space=pl.ANY),
            pl.BlockSpec(memory_space=pl.ANY),
        ],
        out_specs=pl.BlockSpec(memory_space=pltpu.VMEM),
        scratch_shapes=[
            pltpu.VMEM((2, hq, skv, dh), jnp.bfloat16),
            pltpu.VMEM((2, hq, skv, dh), jnp.bfloat16),
            pltpu.VMEM((hq, sq, dh), jnp.bfloat16),
            pltpu.VMEM((hq, sq, dh), jnp.float32),
            pltpu.VMEM((hq, sq, 1), jnp.float32),
            pltpu.VMEM((hq, sq, 1), jnp.float32),
            pltpu.VMEM((d, d), jnp.float32),
            pltpu.SemaphoreType.DMA((2,)),
            pltpu.SemaphoreType.DMA((4,)),
            pltpu.SemaphoreType.DMA((4,)),
            pltpu.SemaphoreType.REGULAR,
            pltpu.SemaphoreType.REGULAR,
        ],
        compiler_params=pltpu.CompilerParams(
            collective_id=0, vmem_limit_bytes=67043328),
    )(x2, Wq, Wo, k3, v3)
    return out.reshape(1, sq, d)
